# baseline (speedup 1.0000x reference)
import jax
import jax.numpy as jnp
from jax import lax
from jax.experimental import pallas as pl
from jax.experimental.pallas import tpu as pltpu
from jax.experimental.pallas import tpu_sc as plsc

B, L, H, OUT, V = 4096, 50, 64, 6, 100000
HP = 128
NC, NS = 2, 16
NW = NC * NS
BPW = B // NW
NB = 4
NG = BPW // NB
HC2 = H // 32
KF = 5


def _sc_body(abs_hbm, tab_hbm, w_hbm, bias_hbm, out_hbm,
             idx_v, w_v, rows_v, out_v, bias_v, tbuf_v, sems):
    wid = lax.axis_index("s") * NC + lax.axis_index("c")
    base = wid * BPW
    pltpu.sync_copy(abs_hbm.at[pl.ds(base, BPW), :], idx_v)
    pltpu.sync_copy(w_hbm, w_v)
    pltpu.sync_copy(bias_hbm.at[pl.ds(0, 16)], bias_v)
    lane = lax.iota(jnp.int32, 16)
    col_idx = [lane * 16 + o for o in range(OUT)]

    def fire(g, phase):
        for b in range(NB):
            pltpu.async_copy(tab_hbm.at[idx_v.at[g * NB + b]],
                             rows_v.at[phase * NB + b], sems.at[phase])

    def drain(g, phase):
        for b in range(NB):
            pltpu.make_async_copy(tab_hbm.at[idx_v.at[g * NB + b]],
                                  rows_v.at[phase * NB + b],
                                  sems.at[phase]).wait()

    fire(0, 0)
    fire(1, 1)

    def pair(i, carry):
        for phase in range(2):
            g = i * 2 + phase
            drain(g, phase)

            def step_body(s, faccs):
                accs = [jnp.zeros((32,), jnp.bfloat16)
                        for _ in range(NB * OUT)]
                for dl in range(KF):
                    l = s * KF + dl
                    rows = [rows_v[phase * NB + b, l, pl.ds(c * 32, 32)]
                            for b in range(NB) for c in range(HC2)]
                    for o in range(OUT):
                        w = [w_v[pl.ds(((l * OUT + o) * H) + c * 32, 32)]
                             for c in range(HC2)]
                        for b in range(NB):
                            a = accs[b * OUT + o]
                            for c in range(HC2):
                                a = a + w[c] * rows[b * HC2 + c]
                            accs[b * OUT + o] = a
                new = list(faccs)
                for idx in range(NB * OUT):
                    p0, p1 = plsc.unpack(
                        accs[idx], format=plsc.PackFormat.INTERLEAVED,
                        preferred_element_type=jnp.float32)
                    new[idx] = new[idx] + (p0 + p1)
                return tuple(new)

            fzeros = tuple(jnp.zeros((16,), jnp.float32)
                           for _ in range(NB * OUT))
            faccs = lax.fori_loop(0, L // KF, step_body, fzeros)

            @pl.when(g + 2 < NG)
            def _():
                fire(g + 2, phase)

            bias = bias_v[:]
            for b in range(NB):
                for o in range(OUT):
                    plsc.store_scatter(tbuf_v, [col_idx[o]],
                                       faccs[b * OUT + o])
                row = bias
                for h in range(16):
                    row = row + tbuf_v[pl.ds(h * 16, 16)]
                out_v[pl.ds((g * NB + b) * 16, 16)] = row
        return carry

    lax.fori_loop(0, NG // 2, pair, 0)
    pltpu.sync_copy(out_v, out_hbm.at[pl.ds(base * 16, BPW * 16)])


@jax.jit
def kernel(abstract, emb_table, W, b):
    tab16 = jnp.pad(emb_table.astype(jnp.bfloat16), ((0, 0), (0, HP - H)))
    wt = (W.reshape(OUT, L, H).transpose(1, 0, 2)
          .astype(jnp.bfloat16).reshape(L * OUT * H))
    bias128 = jnp.zeros((128,), jnp.float32).at[:OUT].set(b)
    mesh = plsc.VectorSubcoreMesh(
        core_axis_name="c", subcore_axis_name="s",
        num_cores=NC, num_subcores=NS)
    f = pl.kernel(
        _sc_body,
        out_type=jax.ShapeDtypeStruct((B * 16,), jnp.float32),
        mesh=mesh,
        compiler_params=pltpu.CompilerParams(
            needs_layout_passes=False, use_tc_tiling_on_sc=False),
        scratch_types=[
            pltpu.VMEM((BPW, L), jnp.int32),
            pltpu.VMEM((L * OUT * H,), jnp.bfloat16),
            pltpu.VMEM((2 * NB, L, HP), jnp.bfloat16),
            pltpu.VMEM((BPW * 16,), jnp.float32),
            pltpu.VMEM((16,), jnp.float32),
            pltpu.VMEM((256,), jnp.float32),
            pltpu.SemaphoreType.DMA((2,)),
        ],
    )
    out = f(abstract, tab16, wt, bias128)
    return out.reshape(B, 16)[:, :OUT]


# speedup vs baseline: 1.4490x; 1.4490x over previous
import jax
import jax.numpy as jnp
from jax import lax
from jax.experimental import pallas as pl
from jax.experimental.pallas import tpu as pltpu
from jax.experimental.pallas import tpu_sc as plsc

B, L, H, OUT, V = 4096, 50, 64, 6, 100000
HP = 128
LP = 56
NC, NS = 2, 16
NW = NC * NS
BPW = B // NW
NB = 4
NG = BPW // NB
HC2 = H // 32
KF = 5


def _sc_body(abs_hbm, tab_hbm, w_hbm, bias_hbm, out_hbm,
             idx_v, w_v, rows_v, out_v, bias_v, tbuf_v, sems):
    wid = lax.axis_index("s") * NC + lax.axis_index("c")
    base = wid * BPW
    pltpu.sync_copy(abs_hbm.at[pl.ds(base * LP, BPW * LP)], idx_v)
    pltpu.sync_copy(w_hbm, w_v)
    pltpu.sync_copy(bias_hbm.at[pl.ds(0, 16)], bias_v)
    lane = lax.iota(jnp.int32, 16)
    col_idx = [lane * 16 + o for o in range(OUT)]

    def fire(g, phase):
        for b in range(NB):
            pltpu.async_copy(
                tab_hbm.at[idx_v.at[pl.ds((g * NB + b) * LP, L)]],
                rows_v.at[phase * NB + b], sems.at[phase])

    def drain(g, phase):
        for b in range(NB):
            pltpu.make_async_copy(
                tab_hbm.at[idx_v.at[pl.ds((g * NB + b) * LP, L)]],
                rows_v.at[phase * NB + b], sems.at[phase]).wait()

    fire(0, 0)
    fire(1, 1)

    def pair(i, carry):
        for phase in range(2):
            g = i * 2 + phase
            drain(g, phase)

            def step_body(s, faccs):
                accs = [jnp.zeros((32,), jnp.bfloat16)
                        for _ in range(NB * OUT)]
                for dl in range(KF):
                    l = s * KF + dl
                    rows = []
                    for b in range(NB):
                        for c in range(HC2):
                            lo = rows_v[phase * NB + b, l, pl.ds(c * 32, 16)]
                            hi = rows_v[phase * NB + b, l,
                                        pl.ds(c * 32 + 16, 16)]
                            rows.append(plsc.pack(
                                lo, hi, format=plsc.PackFormat.INTERLEAVED))
                    for o in range(OUT):
                        w = [plsc.bitcast(
                                w_v[pl.ds((l * OUT + o) * 32 + c * 16, 16)],
                                jnp.bfloat16)
                             for c in range(HC2)]
                        for b in range(NB):
                            a = accs[b * OUT + o]
                            for c in range(HC2):
                                a = a + w[c] * rows[b * HC2 + c]
                            accs[b * OUT + o] = a
                new = list(faccs)
                for k in range(NB * OUT):
                    p0, p1 = plsc.unpack(
                        accs[k], format=plsc.PackFormat.INTERLEAVED,
                        preferred_element_type=jnp.float32)
                    new[k] = new[k] + (p0 + p1)
                return tuple(new)

            fzeros = tuple(jnp.zeros((16,), jnp.float32)
                           for _ in range(NB * OUT))
            faccs = lax.fori_loop(0, L // KF, step_body, fzeros)

            @pl.when(g + 2 < NG)
            def _():
                fire(g + 2, phase)

            bias = bias_v[:]
            for b in range(NB):
                for o in range(OUT):
                    plsc.store_scatter(tbuf_v, [col_idx[o]],
                                       faccs[b * OUT + o])
                row = bias
                for h in range(16):
                    row = row + tbuf_v[pl.ds(h * 16, 16)]
                out_v[pl.ds((g * NB + b) * 16, 16)] = row
        return carry

    lax.fori_loop(0, NG // 2, pair, 0)
    pltpu.sync_copy(out_v, out_hbm.at[pl.ds(base * 16, BPW * 16)])


@jax.jit
def kernel(abstract, emb_table, W, b):
    tabp = jnp.pad(emb_table, ((0, 0), (0, HP - H)))
    wt = (W.reshape(OUT, L, H).transpose(1, 0, 2)
          .reshape(L, OUT, HC2, 2, 16).transpose(0, 1, 2, 4, 3)
          .astype(jnp.bfloat16).reshape(L * OUT * H // 2, 2))
    w_pk = jax.lax.bitcast_convert_type(wt, jnp.float32)
    ab_flat = jnp.pad(abstract, ((0, 0), (0, LP - L))).reshape(B * LP)
    bias128 = jnp.zeros((128,), jnp.float32).at[:OUT].set(b)
    mesh = plsc.VectorSubcoreMesh(
        core_axis_name="c", subcore_axis_name="s",
        num_cores=NC, num_subcores=NS)
    f = pl.kernel(
        _sc_body,
        out_type=jax.ShapeDtypeStruct((B * 16,), jnp.float32),
        mesh=mesh,
        compiler_params=pltpu.CompilerParams(
            needs_layout_passes=False, use_tc_tiling_on_sc=False),
        scratch_types=[
            pltpu.VMEM((BPW * LP,), jnp.int32),
            pltpu.VMEM((L * OUT * H // 2,), jnp.float32),
            pltpu.VMEM((2 * NB, L, HP), jnp.float32),
            pltpu.VMEM((BPW * 16,), jnp.float32),
            pltpu.VMEM((16,), jnp.float32),
            pltpu.VMEM((256,), jnp.float32),
            pltpu.SemaphoreType.DMA((2,)),
        ],
    )
    out = f(ab_flat, tabp, w_pk, bias128)
    return out.reshape(B, 16)[:, :OUT]


# speedup vs baseline: 1.8483x; 1.2756x over previous
import jax
import jax.numpy as jnp
from jax import lax
from jax.experimental import pallas as pl
from jax.experimental.pallas import tpu as pltpu
from jax.experimental.pallas import tpu_sc as plsc

B, L, H, OUT, V = 4096, 50, 64, 6, 100000
HP = H
LP = 56
NC, NS = 2, 16
NW = NC * NS
BPW = B // NW
NB = 4
NG = BPW // NB
HC2 = H // 32
KF = 10


def _sc_body(abs_hbm, tab_hbm, w_hbm, bias_hbm, out_hbm,
             idx_v, w_v, rows_v, out_v, bias_v, tbuf_v, sems):
    wid = lax.axis_index("s") * NC + lax.axis_index("c")
    base = wid * BPW
    pltpu.sync_copy(abs_hbm.at[pl.ds(base * LP, BPW * LP)], idx_v)
    pltpu.sync_copy(w_hbm, w_v)
    pltpu.sync_copy(bias_hbm.at[pl.ds(0, 16)], bias_v)
    lane = lax.iota(jnp.int32, 16)
    col_idx = [lane * 16 + o for o in range(OUT)]

    def fire(g, phase):
        for b in range(NB):
            pltpu.async_copy(
                tab_hbm.at[idx_v.at[pl.ds((g * NB + b) * LP, L)]],
                rows_v.at[phase * NB + b], sems.at[phase])

    def drain(g, phase):
        for b in range(NB):
            pltpu.make_async_copy(
                tab_hbm.at[idx_v.at[pl.ds((g * NB + b) * LP, L)]],
                rows_v.at[phase * NB + b], sems.at[phase]).wait()

    fire(0, 0)
    fire(1, 1)

    def group_body(g, carry):
        phase = lax.rem(g, 2)
        pb = phase * NB

        @pl.when(phase == 0)
        def _():
            drain(g, 0)

        @pl.when(phase == 1)
        def _():
            drain(g, 1)

        def step_body(s, faccs):
            accs = [jnp.zeros((32,), jnp.bfloat16)
                    for _ in range(NB * OUT)]
            for dl in range(KF):
                l = s * KF + dl
                rows = []
                for b in range(NB):
                    for c in range(HC2):
                        lo = rows_v[pb + b, l, pl.ds(c * 32, 16)]
                        hi = rows_v[pb + b, l, pl.ds(c * 32 + 16, 16)]
                        rows.append(plsc.pack(
                            lo, hi, format=plsc.PackFormat.INTERLEAVED))
                for o in range(OUT):
                    w = [plsc.bitcast(
                            w_v[pl.ds((l * OUT + o) * 32 + c * 16, 16)],
                            jnp.bfloat16)
                         for c in range(HC2)]
                    for b in range(NB):
                        a = accs[b * OUT + o]
                        for c in range(HC2):
                            a = a + w[c] * rows[b * HC2 + c]
                        accs[b * OUT + o] = a
            new = list(faccs)
            for k in range(NB * OUT):
                p0, p1 = plsc.unpack(
                    accs[k], format=plsc.PackFormat.INTERLEAVED,
                    preferred_element_type=jnp.float32)
                new[k] = new[k] + (p0 + p1)
            return tuple(new)

        fzeros = tuple(jnp.zeros((16,), jnp.float32)
                       for _ in range(NB * OUT))
        faccs = lax.fori_loop(0, L // KF, step_body, fzeros)

        @pl.when(jnp.logical_and(g + 2 < NG, phase == 0))
        def _():
            fire(g + 2, 0)

        @pl.when(jnp.logical_and(g + 2 < NG, phase == 1))
        def _():
            fire(g + 2, 1)

        bias = bias_v[:]
        for b in range(NB):
            for o in range(OUT):
                plsc.store_scatter(tbuf_v, [col_idx[o]],
                                   faccs[b * OUT + o])
            row = bias
            for h in range(16):
                row = row + tbuf_v[pl.ds(h * 16, 16)]
            out_v[pl.ds((g * NB + b) * 16, 16)] = row
        return carry

    lax.fori_loop(0, NG, group_body, 0)
    pltpu.sync_copy(out_v, out_hbm.at[pl.ds(base * 16, BPW * 16)])


@jax.jit
def kernel(abstract, emb_table, W, b):
    tabp = emb_table
    wt = (W.reshape(OUT, L, H).transpose(1, 0, 2)
          .reshape(L, OUT, HC2, 2, 16).transpose(0, 1, 2, 4, 3)
          .astype(jnp.bfloat16).reshape(L * OUT * H // 2, 2))
    w_pk = jax.lax.bitcast_convert_type(wt, jnp.float32)
    ab_flat = jnp.pad(abstract, ((0, 0), (0, LP - L))).reshape(B * LP)
    bias128 = jnp.zeros((128,), jnp.float32).at[:OUT].set(b)
    mesh = plsc.VectorSubcoreMesh(
        core_axis_name="c", subcore_axis_name="s",
        num_cores=NC, num_subcores=NS)
    f = pl.kernel(
        _sc_body,
        out_type=jax.ShapeDtypeStruct((B * 16,), jnp.float32),
        mesh=mesh,
        compiler_params=pltpu.CompilerParams(
            needs_layout_passes=False, use_tc_tiling_on_sc=False),
        scratch_types=[
            pltpu.VMEM((BPW * LP,), jnp.int32),
            pltpu.VMEM((L * OUT * H // 2,), jnp.float32),
            pltpu.VMEM((2 * NB, L, HP), jnp.float32),
            pltpu.VMEM((BPW * 16,), jnp.float32),
            pltpu.VMEM((16,), jnp.float32),
            pltpu.VMEM((256,), jnp.float32),
            pltpu.SemaphoreType.DMA((2,)),
        ],
    )
    out = f(ab_flat, tabp, w_pk, bias128)
    return out.reshape(B, 16)[:, :OUT]
